# baseline (speedup 1.0000x reference)
import functools

import jax
import jax.numpy as jnp
from jax import lax
from jax.experimental import pallas as pl
from jax.experimental.pallas import tpu as pltpu
from jax.experimental.pallas import tpu_sc as plsc

_TOKENS = 32768
_E = 64
_K = 8
_NW = 32
_PT = _TOKENS // _NW
_CH = 256
_NCH = _PT // _CH


def _mm_block(h_ref, wt_ref, b_ref, out_ref):
    out_ref[...] = (
        jnp.dot(h_ref[...], wt_ref[...], preferred_element_type=jnp.float32)
        + b_ref[...]
    )


def _tc_logits(hidden_states, wt, b2):
    t, h = hidden_states.shape
    e = wt.shape[1]
    bt = 512
    return pl.pallas_call(
        _mm_block,
        grid=(t // bt,),
        in_specs=[
            pl.BlockSpec((bt, h), lambda i: (i, 0)),
            pl.BlockSpec((h, e), lambda i: (0, 0)),
            pl.BlockSpec((1, e), lambda i: (0, 0)),
        ],
        out_specs=pl.BlockSpec((bt, e), lambda i: (i, 0)),
        out_shape=jax.ShapeDtypeStruct((t, e), jnp.float32),
    )(hidden_states, wt, b2)


@functools.partial(
    pl.kernel,
    out_type=[
        jax.ShapeDtypeStruct((_TOKENS, _E), jnp.float32),
        jax.ShapeDtypeStruct((_TOKENS, _K), jnp.int32),
    ],
    mesh=plsc.VectorSubcoreMesh(core_axis_name="c", subcore_axis_name="s"),
    compiler_params=pltpu.CompilerParams(needs_layout_passes=False),
    scratch_types=[
        pltpu.VMEM((_CH, _E), jnp.float32),
        pltpu.VMEM((_CH, _E), jnp.float32),
        pltpu.VMEM((_CH, _K), jnp.int32),
        pltpu.VMEM((32,), jnp.float32),
        pltpu.VMEM((32,), jnp.int32),
    ],
)
def _sc_topk(lg_hbm, scores_hbm, idx_hbm, in_v, sc_v, ix_v, mg_v, mg_i):
    wid = lax.axis_index("s") * 2 + lax.axis_index("c")
    iota = lax.iota(jnp.int32, 16)
    fiota = iota.astype(jnp.float32)
    in_lo = iota < 8
    gsel = jnp.where(in_lo, iota, iota + 8)

    for c in range(_NCH):
        base = wid * _PT + c * _CH
        pltpu.sync_copy(lg_hbm.at[pl.ds(base, _CH), :], in_v)

        def tok_body(t, _):
            v0 = in_v[t, pl.ds(0, 16)]
            v1 = in_v[t, pl.ds(16, 16)]
            v2 = in_v[t, pl.ds(32, 16)]
            v3 = in_v[t, pl.ds(48, 16)]

            sv0, si0 = plsc.sort_key_val(v0, iota, descending=True)
            sv1, si1 = plsc.sort_key_val(v1, iota + 16, descending=True)
            sv2, si2 = plsc.sort_key_val(v2, iota + 32, descending=True)
            sv3, si3 = plsc.sort_key_val(v3, iota + 48, descending=True)

            def merge(av, ai, bv, bi):
                mg_v[pl.ds(0, 16)] = av
                mg_v[pl.ds(16, 16)] = bv
                mg_i[pl.ds(0, 16)] = ai
                mg_i[pl.ds(16, 16)] = bi
                cv = plsc.load_gather(mg_v, [gsel])
                ci = plsc.load_gather(mg_i, [gsel])
                return plsc.sort_key_val(cv, ci, descending=True)

            mv01, mi01 = merge(sv0, si0, sv1, si1)
            mv23, mi23 = merge(sv2, si2, sv3, si3)
            fv, fi = merge(mv01, mi01, mv23, mi23)

            vmax = lax.reduce_max(fv, axes=(0,))
            mg_v[pl.ds(0, 16)] = fv
            v8 = plsc.load_gather(mg_v, [jnp.full((16,), 7, jnp.int32)])

            ex8 = jnp.exp(fv - vmax) * jnp.where(in_lo, 1.0, 0.0)
            den = lax.reduce_sum(ex8, axes=(0,))
            rden = 1.0 / (jnp.zeros((16,), jnp.float32) + den)

            for k, vk in enumerate((v0, v1, v2, v3)):
                sc_v[t, pl.ds(k * 16, 16)] = jnp.where(
                    vk >= v8, jnp.exp(vk - vmax) * rden, 0.0
                )
            plsc.store_scatter(
                ix_v, [jnp.full((16,), 0, jnp.int32) + t, iota], fi,
                mask=in_lo,
            )
            return _

        lax.fori_loop(0, _CH, tok_body, 0)

        pltpu.sync_copy(sc_v, scores_hbm.at[pl.ds(base, _CH), :])
        pltpu.sync_copy(ix_v, idx_hbm.at[pl.ds(base, _CH), :])


@functools.partial(jax.jit, static_argnames=())
def kernel(hidden_states, weight, bias):
    e = weight.shape[0]
    wt = weight.T
    b2 = bias.reshape(1, e)
    logits = _tc_logits(hidden_states, wt, b2)
    scores, idx = _sc_topk(logits)
    return scores, idx


# speedup vs baseline: 1.1076x; 1.1076x over previous
import functools

import jax
import jax.numpy as jnp
from jax.experimental import pallas as pl
from jax.experimental.pallas import tpu as pltpu

_TOP_K = 8
_BT = 256


def _topk_block(logits):
    bt, e = logits.shape
    neg = jnp.float32(-3.0e38)
    zero = jnp.zeros((), jnp.float32)
    coef = (262144.0, 4096.0, 64.0, 1.0)

    vals = logits
    s_lo = jnp.zeros((bt, e), jnp.float32)
    s_hi = jnp.zeros((bt, e), jnp.float32)
    vmax = None
    den = None
    v8 = None
    for j in range(_TOP_K):
        m = jnp.max(vals, axis=1, keepdims=True)
        eq = vals == m
        if j == 0:
            vmax = m
            den = jnp.ones((bt, 1), jnp.float32)
        else:
            den = den + jnp.exp(m - vmax)
        if j == _TOP_K - 1:
            v8 = m
        if j < 4:
            s_lo = s_lo + jnp.where(eq, coef[j], zero)
        else:
            s_hi = s_hi + jnp.where(eq, coef[j - 4], zero)
        if j < _TOP_K - 1:
            vals = jnp.where(eq, neg, vals)

    rden = 1.0 / den
    scores = jnp.where(logits >= v8, jnp.exp(logits - vmax) * rden, 0.0)

    lane = jax.lax.broadcasted_iota(jnp.int32, (bt, e), 1).astype(jnp.float32)
    w_lo = s_lo * lane
    w_hi = s_hi * lane

    wcat = jnp.concatenate([w_lo, w_hi], axis=1)
    sel_lo = (jax.lax.broadcasted_iota(jnp.int32, (2 * e, 2), 0) < e)
    sel = jnp.where(
        sel_lo == (jax.lax.broadcasted_iota(jnp.int32, (2 * e, 2), 1) == 0),
        1.0, 0.0,
    ).astype(jnp.float32)
    packed = jnp.dot(wcat, sel, preferred_element_type=jnp.float32)
    p_lo = packed[:, :1]
    p_hi = packed[:, 1:2]

    digits = []
    for p in (p_lo, p_hi):
        d0 = jnp.floor(p * (1.0 / 262144.0))
        r0 = p - d0 * 262144.0
        d1 = jnp.floor(r0 * (1.0 / 4096.0))
        r1 = r0 - d1 * 4096.0
        d2 = jnp.floor(r1 * (1.0 / 64.0))
        d3 = r1 - d2 * 64.0
        digits += [d0, d1, d2, d3]
    idx = jnp.concatenate(digits, axis=1).astype(jnp.int32)
    return scores, idx


def _router_block(h_ref, wt_ref, b_ref, scores_ref, idx_ref, lg_ref):
    lg_prev = lg_ref[...]
    scores, idx = _topk_block(lg_prev)
    scores_ref[...] = scores
    idx_ref[...] = idx
    lg_ref[...] = (
        jnp.dot(h_ref[...], wt_ref[...], preferred_element_type=jnp.float32)
        + b_ref[...]
    )


@functools.partial(jax.jit, static_argnames=())
def kernel(hidden_states, weight, bias):
    t, h = hidden_states.shape
    e = weight.shape[0]
    nblk = t // _BT
    grid = (nblk + 1,)

    wt = weight.T
    b2 = bias.reshape(1, e)

    scores, idx = pl.pallas_call(
        _router_block,
        grid=grid,
        in_specs=[
            pl.BlockSpec((_BT, h), lambda i: (jnp.minimum(i, nblk - 1), 0)),
            pl.BlockSpec((h, e), lambda i: (0, 0)),
            pl.BlockSpec((1, e), lambda i: (0, 0)),
        ],
        out_specs=[
            pl.BlockSpec((_BT, e), lambda i: (jnp.maximum(i - 1, 0), 0)),
            pl.BlockSpec((_BT, _TOP_K), lambda i: (jnp.maximum(i - 1, 0), 0)),
        ],
        out_shape=[
            jax.ShapeDtypeStruct((t, e), jnp.float32),
            jax.ShapeDtypeStruct((t, _TOP_K), jnp.int32),
        ],
        scratch_shapes=[pltpu.VMEM((_BT, e), jnp.float32)],
    )(hidden_states, wt, b2)
    return scores, idx


# speedup vs baseline: 1.3163x; 1.1884x over previous
import functools

import jax
import jax.numpy as jnp
from jax.experimental import pallas as pl
from jax.experimental.pallas import tpu as pltpu

_TOP_K = 8
_BSUB = 256
_NSUB = 2
_BT = _BSUB * _NSUB


def _topk_block(logits):
    bt, e = logits.shape
    neg = jnp.float32(-3.0e38)
    zero = jnp.zeros((), jnp.float32)
    coef = (262144.0, 4096.0, 64.0, 1.0)

    vals = logits
    s_lo = jnp.zeros((bt, e), jnp.float32)
    s_hi = jnp.zeros((bt, e), jnp.float32)
    vmax = None
    den = None
    v8 = None
    for j in range(_TOP_K):
        m = jnp.max(vals, axis=1, keepdims=True)
        eq = vals == m
        if j == 0:
            vmax = m
            den = jnp.ones((bt, 1), jnp.float32)
        else:
            den = den + jnp.exp(m - vmax)
        if j == _TOP_K - 1:
            v8 = m
        if j < 4:
            s_lo = s_lo + jnp.where(eq, coef[j], zero)
        else:
            s_hi = s_hi + jnp.where(eq, coef[j - 4], zero)
        if j < _TOP_K - 1:
            vals = jnp.where(eq, neg, vals)

    rden = 1.0 / den
    scores = jnp.where(logits >= v8, jnp.exp(logits - vmax) * rden, 0.0)

    lane = jax.lax.broadcasted_iota(jnp.int32, (bt, e), 1).astype(jnp.float32)
    w_lo = s_lo * lane
    w_hi = s_hi * lane

    wcat = jnp.concatenate([w_lo, w_hi], axis=1)
    sel_lo = (jax.lax.broadcasted_iota(jnp.int32, (2 * e, 2), 0) < e)
    sel = jnp.where(
        sel_lo == (jax.lax.broadcasted_iota(jnp.int32, (2 * e, 2), 1) == 0),
        1.0, 0.0,
    ).astype(jnp.float32)
    packed = jnp.dot(wcat, sel, preferred_element_type=jnp.float32)
    p_lo = packed[:, :1]
    p_hi = packed[:, 1:2]

    digits = []
    for p in (p_lo, p_hi):
        d0 = jnp.floor(p * (1.0 / 262144.0))
        r0 = p - d0 * 262144.0
        d1 = jnp.floor(r0 * (1.0 / 4096.0))
        r1 = r0 - d1 * 4096.0
        d2 = jnp.floor(r1 * (1.0 / 64.0))
        d3 = r1 - d2 * 64.0
        digits += [d0, d1, d2, d3]
    idx = jnp.concatenate(digits, axis=1).astype(jnp.int32)
    return scores, idx


def _router_block(h_ref, wt_ref, b_ref, scores_ref, idx_ref, lg_ref):
    i = pl.program_id(0)
    nsteps = pl.num_programs(0)
    wt = wt_ref[...]
    b2 = b_ref[...]

    def dot_sub(s):
        h = h_ref[pl.ds(s * _BSUB, _BSUB), :]
        return jnp.dot(h, wt, preferred_element_type=jnp.float32) + b2

    def emit(row_start, lg):
        scores, idx = _topk_block(lg)
        scores_ref[pl.ds(row_start, _BSUB), :] = scores
        idx_ref[pl.ds(row_start, _BSUB), :] = idx

    base = i * _BT
    carry_start = jnp.maximum(base - _BSUB, 0)
    lg = lg_ref[...]
    for s in range(_NSUB):
        lg_next = dot_sub(s)
        emit(carry_start if s == 0 else base + (s - 1) * _BSUB, lg)
        lg = lg_next
    lg_ref[...] = lg

    @pl.when(i == nsteps - 1)
    def _drain():
        emit(base + (_NSUB - 1) * _BSUB, lg)


@functools.partial(jax.jit, static_argnames=())
def kernel(hidden_states, weight, bias):
    t, h = hidden_states.shape
    e = weight.shape[0]
    nblk = t // _BT
    grid = (nblk,)

    wt = weight.T
    b2 = bias.reshape(1, e)

    scores, idx = pl.pallas_call(
        _router_block,
        grid=grid,
        in_specs=[
            pl.BlockSpec((_BT, h), lambda i: (i, 0)),
            pl.BlockSpec((h, e), lambda i: (0, 0)),
            pl.BlockSpec((1, e), lambda i: (0, 0)),
        ],
        out_specs=[
            pl.BlockSpec((t, e), lambda i: (0, 0)),
            pl.BlockSpec((t, _TOP_K), lambda i: (0, 0)),
        ],
        out_shape=[
            jax.ShapeDtypeStruct((t, e), jnp.float32),
            jax.ShapeDtypeStruct((t, _TOP_K), jnp.int32),
        ],
        scratch_shapes=[pltpu.VMEM((_BSUB, e), jnp.float32)],
    )(hidden_states, wt, b2)
    return scores, idx


# speedup vs baseline: 1.4204x; 1.0791x over previous
import functools

import jax
import jax.numpy as jnp
from jax.experimental import pallas as pl

_TOP_K = 8
_BSUB = 256
_NSUB = 4


def _topk_block(logits):
    bt, e = logits.shape
    neg = jnp.float32(-3.0e38)
    zero = jnp.zeros((), jnp.float32)
    coef = (262144.0, 4096.0, 64.0, 1.0)

    vals = logits
    s_lo = jnp.zeros((bt, e), jnp.float32)
    s_hi = jnp.zeros((bt, e), jnp.float32)
    vmax = None
    den = None
    v8 = None
    for j in range(_TOP_K):
        m = jnp.max(vals, axis=1, keepdims=True)
        eq = vals == m
        if j == 0:
            vmax = m
            den = jnp.ones((bt, 1), jnp.float32)
        else:
            den = den + jnp.exp(m - vmax)
        if j == _TOP_K - 1:
            v8 = m
        if j < 4:
            s_lo = s_lo + jnp.where(eq, coef[j], zero)
        else:
            s_hi = s_hi + jnp.where(eq, coef[j - 4], zero)
        if j < _TOP_K - 1:
            vals = jnp.where(eq, neg, vals)

    rden = 1.0 / den
    scores = jnp.where(logits >= v8, jnp.exp(logits - vmax) * rden, 0.0)

    lane = jax.lax.broadcasted_iota(jnp.int32, (bt, e), 1).astype(jnp.float32)
    w_lo = s_lo * lane
    w_hi = s_hi * lane

    wcat = jnp.concatenate([w_lo, w_hi], axis=1)
    sel_lo = (jax.lax.broadcasted_iota(jnp.int32, (2 * e, 2), 0) < e)
    sel = jnp.where(
        sel_lo == (jax.lax.broadcasted_iota(jnp.int32, (2 * e, 2), 1) == 0),
        1.0, 0.0,
    ).astype(jnp.float32)
    packed = jnp.dot(wcat, sel, preferred_element_type=jnp.float32)
    p_lo = packed[:, :1]
    p_hi = packed[:, 1:2]

    digits = []
    for p in (p_lo, p_hi):
        d0 = jnp.floor(p * (1.0 / 262144.0))
        r0 = p - d0 * 262144.0
        d1 = jnp.floor(r0 * (1.0 / 4096.0))
        r1 = r0 - d1 * 4096.0
        d2 = jnp.floor(r1 * (1.0 / 64.0))
        d3 = r1 - d2 * 64.0
        digits += [d0, d1, d2, d3]
    idx = jnp.concatenate(digits, axis=1).astype(jnp.int32)
    return scores, idx


def _router_block(h_ref, wt_ref, b_ref, scores_ref, idx_ref):
    wt = wt_ref[...]
    b2 = b_ref[...]

    def dot_sub(s):
        h = h_ref[pl.ds(s * _BSUB, _BSUB), :]
        return jnp.dot(h, wt, preferred_element_type=jnp.float32) + b2

    def emit_sub(s, lg):
        scores, idx = _topk_block(lg)
        scores_ref[pl.ds(s * _BSUB, _BSUB), :] = scores
        idx_ref[pl.ds(s * _BSUB, _BSUB), :] = idx

    lg = dot_sub(0)
    for s in range(_NSUB):
        lg_next = dot_sub(s + 1) if s + 1 < _NSUB else None
        emit_sub(s, lg)
        lg = lg_next


@functools.partial(jax.jit, static_argnames=())
def kernel(hidden_states, weight, bias):
    t, h = hidden_states.shape
    e = weight.shape[0]
    bt = _BSUB * _NSUB
    grid = (t // bt,)

    wt = weight.T
    b2 = bias.reshape(1, e)

    scores, idx = pl.pallas_call(
        _router_block,
        grid=grid,
        in_specs=[
            pl.BlockSpec((bt, h), lambda i: (i, 0)),
            pl.BlockSpec((h, e), lambda i: (0, 0)),
            pl.BlockSpec((1, e), lambda i: (0, 0)),
        ],
        out_specs=[
            pl.BlockSpec((bt, e), lambda i: (i, 0)),
            pl.BlockSpec((bt, _TOP_K), lambda i: (i, 0)),
        ],
        out_shape=[
            jax.ShapeDtypeStruct((t, e), jnp.float32),
            jax.ShapeDtypeStruct((t, _TOP_K), jnp.int32),
        ],
    )(hidden_states, wt, b2)
    return scores, idx
